# baseline (device time: 177919 ns/iter reference)
import jax
import jax.numpy as jnp
from jax import lax
from jax.experimental import pallas as pl
from jax.experimental.pallas import tpu as pltpu

N_DEV = 16


def kernel(x, Win0, Wout0, Win1, Wout1, Win2, Wout2):
    B, D = x.shape
    C = B // N_DEV

    def body(x_ref, win0_ref, wout0_ref, win1_ref, wout1_ref,
             win2_ref, wout2_ref, out_ref,
             xbuf, pbuf, csend, crecv, send_sems, recv_sems):
        i = lax.axis_index("i")
        left = (i - 1) % N_DEV
        right = (i + 1) % N_DEV

        barrier_sem = pltpu.get_barrier_semaphore()
        for nbr in (left, right):
            pl.semaphore_signal(barrier_sem, inc=1, device_id=(nbr,),
                                device_id_type=pl.DeviceIdType.MESH)
        pl.semaphore_wait(barrier_sem, 2)

        hop = [0]

        def ring_hop(val):
            slot = hop[0] % N_DEV
            hop[0] += 1
            csend[slot] = val
            rdma = pltpu.make_async_remote_copy(
                src_ref=csend.at[slot],
                dst_ref=crecv.at[slot],
                send_sem=send_sems.at[slot],
                recv_sem=recv_sems.at[slot],
                device_id=(right,),
                device_id_type=pl.DeviceIdType.MESH,
            )
            rdma.start()
            rdma.wait()
            return crecv[slot]

        wins = (win0_ref, win1_ref, win2_ref)
        wouts = (wout0_ref, wout1_ref, wout2_ref)

        xv = x_ref[...]
        for l in range(3):
            h = jnp.maximum(
                jnp.dot(xv, wins[l][...], preferred_element_type=jnp.float32),
                0.0)
            pbuf[...] = jnp.dot(h, wouts[l][...],
                                preferred_element_type=jnp.float32)

            c = (i - 1) % N_DEV
            acc = pbuf[pl.ds(c * C, C), :]
            for s in range(N_DEV - 1):
                recv = ring_hop(acc)
                c = (i - s - 2) % N_DEV
                acc = recv + pbuf[pl.ds(c * C, C), :]

            if l < 2:
                xbuf[pl.ds(i * C, C), :] = acc
                cur = acc
                for s in range(N_DEV - 1):
                    cur = ring_hop(cur)
                    src_pos = (i - 1 - s) % N_DEV
                    xbuf[pl.ds(src_pos * C, C), :] = cur
                xv = xbuf[...]
            else:
                out_ref[...] = acc

    return pl.pallas_call(
        body,
        out_shape=jax.ShapeDtypeStruct((C, D), jnp.float32),
        in_specs=[pl.BlockSpec(memory_space=pltpu.VMEM)] * 7,
        out_specs=pl.BlockSpec(memory_space=pltpu.VMEM),
        scratch_shapes=[
            pltpu.VMEM((B, D), jnp.float32),
            pltpu.VMEM((B, D), jnp.float32),
            pltpu.VMEM((N_DEV, C, D), jnp.float32),
            pltpu.VMEM((N_DEV, C, D), jnp.float32),
            pltpu.SemaphoreType.DMA((N_DEV,)),
            pltpu.SemaphoreType.DMA((N_DEV,)),
        ],
        compiler_params=pltpu.CompilerParams(collective_id=0),
    )(x, Win0, Wout0, Win1, Wout1, Win2, Wout2)


# device time: 52214 ns/iter; 3.4075x vs baseline; 3.4075x over previous
import jax
import jax.numpy as jnp
from jax import lax
from jax.experimental import pallas as pl
from jax.experimental.pallas import tpu as pltpu

N_DEV = 16


def kernel(x, Win0, Wout0, Win1, Wout1, Win2, Wout2):
    B, D = x.shape
    C = B // N_DEV

    def body(x_ref, win0_ref, wout0_ref, win1_ref, wout1_ref,
             win2_ref, wout2_ref, out_ref,
             xbuf, pbuf, crecv, send_a, recv_a, send_b, recv_b):
        i = lax.axis_index("i")

        barrier_sem = pltpu.get_barrier_semaphore()
        for o in range(1, N_DEV):
            pl.semaphore_signal(barrier_sem, inc=1,
                                device_id=((i + o) % N_DEV,),
                                device_id_type=pl.DeviceIdType.MESH)
        pl.semaphore_wait(barrier_sem, N_DEV - 1)

        phase = [0]

        def sems():
            ab = phase[0] % 2
            phase[0] += 1
            return (send_a, recv_a) if ab == 0 else (send_b, recv_b)

        def reduce_scatter():
            ss, rs = sems()
            descs = []
            for o in range(1, N_DEV):
                t = (i + o) % N_DEV
                d = pltpu.make_async_remote_copy(
                    src_ref=pbuf.at[pl.ds(t * C, C), :],
                    dst_ref=crecv.at[o],
                    send_sem=ss.at[o],
                    recv_sem=rs.at[o],
                    device_id=(t,),
                    device_id_type=pl.DeviceIdType.MESH,
                )
                d.start()
                descs.append(d)
            acc = pbuf[pl.ds(i * C, C), :]
            for o, d in zip(range(1, N_DEV), descs):
                d.wait_recv()
                acc = acc + crecv[o]
            for d in descs:
                d.wait_send()
            return acc

        def all_gather(acc):
            ss, rs = sems()
            xbuf[pl.ds(i * C, C), :] = acc
            descs = []
            for o in range(1, N_DEV):
                t = (i + o) % N_DEV
                d = pltpu.make_async_remote_copy(
                    src_ref=xbuf.at[pl.ds(i * C, C), :],
                    dst_ref=xbuf.at[pl.ds(i * C, C), :],
                    send_sem=ss.at[o],
                    recv_sem=rs.at[o],
                    device_id=(t,),
                    device_id_type=pl.DeviceIdType.MESH,
                )
                d.start()
                descs.append(d)
            for o in range(1, N_DEV):
                src = (i - o) % N_DEV
                w = pltpu.make_async_remote_copy(
                    src_ref=xbuf.at[pl.ds(src * C, C), :],
                    dst_ref=xbuf.at[pl.ds(src * C, C), :],
                    send_sem=ss.at[o],
                    recv_sem=rs.at[o],
                    device_id=(src,),
                    device_id_type=pl.DeviceIdType.MESH,
                )
                w.wait_recv()
            for d in descs:
                d.wait_send()
            return xbuf[...]

        wins = (win0_ref, win1_ref, win2_ref)
        wouts = (wout0_ref, wout1_ref, wout2_ref)

        xv = x_ref[...]
        for l in range(3):
            h = jnp.maximum(
                jnp.dot(xv, wins[l][...], preferred_element_type=jnp.float32),
                0.0)
            pbuf[...] = jnp.dot(h, wouts[l][...],
                                preferred_element_type=jnp.float32)
            acc = reduce_scatter()
            if l < 2:
                xv = all_gather(acc)
            else:
                out_ref[...] = acc

    return pl.pallas_call(
        body,
        out_shape=jax.ShapeDtypeStruct((C, D), jnp.float32),
        in_specs=[pl.BlockSpec(memory_space=pltpu.VMEM)] * 7,
        out_specs=pl.BlockSpec(memory_space=pltpu.VMEM),
        scratch_shapes=[
            pltpu.VMEM((B, D), jnp.float32),
            pltpu.VMEM((B, D), jnp.float32),
            pltpu.VMEM((N_DEV, C, D), jnp.float32),
            pltpu.SemaphoreType.DMA((N_DEV,)),
            pltpu.SemaphoreType.DMA((N_DEV,)),
            pltpu.SemaphoreType.DMA((N_DEV,)),
            pltpu.SemaphoreType.DMA((N_DEV,)),
        ],
        compiler_params=pltpu.CompilerParams(collective_id=0),
    )(x, Win0, Wout0, Win1, Wout1, Win2, Wout2)


# device time: 52131 ns/iter; 3.4129x vs baseline; 1.0016x over previous
import jax
import jax.numpy as jnp
from jax import lax
from jax.experimental import pallas as pl
from jax.experimental.pallas import tpu as pltpu

N_DEV = 16
HALF = N_DEV // 2


def kernel(x, Win0, Wout0, Win1, Wout1, Win2, Wout2):
    B, D = x.shape
    C = B // N_DEV

    def body(x_ref, win0_ref, wout0_ref, win1_ref, wout1_ref,
             win2_ref, wout2_ref, out_ref,
             xbuf, pbuf, crecv, send_a, recv_a, send_b, recv_b):
        i = lax.axis_index("i")
        f32 = jnp.float32

        barrier_sem = pltpu.get_barrier_semaphore()
        for o in range(1, N_DEV):
            pl.semaphore_signal(barrier_sem, inc=1,
                                device_id=((i + o) % N_DEV,),
                                device_id_type=pl.DeviceIdType.MESH)
        pl.semaphore_wait(barrier_sem, N_DEV - 1)

        for q in range(N_DEV):
            xbuf[pl.ds(q * C, C), :] = x_ref[
                pl.ds(((i - q) % N_DEV) * C, C), :]

        def compute_block(win_ref, wout_ref, q0, q1):
            rows = pl.ds(q0 * C, (q1 - q0) * C)
            h = jnp.maximum(
                jnp.dot(xbuf[rows, :], win_ref[...],
                        preferred_element_type=f32), 0.0)
            pbuf[rows, :] = jnp.dot(h, wout_ref[...],
                                    preferred_element_type=f32)

        def rs_send(p):
            d = pltpu.make_async_remote_copy(
                src_ref=pbuf.at[pl.ds(p * C, C), :],
                dst_ref=crecv.at[p],
                send_sem=send_a.at[p],
                recv_sem=recv_a.at[p],
                device_id=((i - p) % N_DEV,),
                device_id_type=pl.DeviceIdType.MESH,
            )
            d.start()
            return d

        def ag_send(o):
            d = pltpu.make_async_remote_copy(
                src_ref=xbuf.at[pl.ds(0, C), :],
                dst_ref=xbuf.at[pl.ds(o * C, C), :],
                send_sem=send_b.at[o],
                recv_sem=recv_b.at[o],
                device_id=((i + o) % N_DEV,),
                device_id_type=pl.DeviceIdType.MESH,
            )
            d.start()
            return d

        def ag_wait_recv(o):
            w = pltpu.make_async_remote_copy(
                src_ref=xbuf.at[pl.ds(o * C, C), :],
                dst_ref=xbuf.at[pl.ds(o * C, C), :],
                send_sem=send_b.at[o],
                recv_sem=recv_b.at[o],
                device_id=(i,),
                device_id_type=pl.DeviceIdType.MESH,
            )
            w.wait_recv()

        wins = (win0_ref, win1_ref, win2_ref)
        wouts = (wout0_ref, wout1_ref, wout2_ref)

        ag_descs = []
        for l in range(3):
            if l > 0:
                for o in range(1, HALF):
                    ag_wait_recv(o)
            compute_block(wins[l], wouts[l], 0, HALF)
            rs_descs = [rs_send(p) for p in range(1, HALF)]
            if l > 0:
                for o in range(HALF, N_DEV):
                    ag_wait_recv(o)
            compute_block(wins[l], wouts[l], HALF, N_DEV)
            rs_descs += [rs_send(p) for p in range(HALF, N_DEV)]
            for d in ag_descs:
                d.wait_send()

            acc = pbuf[pl.ds(0, C), :]
            for p, d in zip(range(1, N_DEV), rs_descs):
                d.wait_recv()
                acc = acc + crecv[p]
            for d in rs_descs:
                d.wait_send()

            if l < 2:
                xbuf[pl.ds(0, C), :] = acc
                ag_descs = [ag_send(o) for o in range(1, N_DEV)]
            else:
                out_ref[...] = acc

    return pl.pallas_call(
        body,
        out_shape=jax.ShapeDtypeStruct((C, D), jnp.float32),
        in_specs=[pl.BlockSpec(memory_space=pltpu.VMEM)] * 7,
        out_specs=pl.BlockSpec(memory_space=pltpu.VMEM),
        scratch_shapes=[
            pltpu.VMEM((B, D), jnp.float32),
            pltpu.VMEM((B, D), jnp.float32),
            pltpu.VMEM((N_DEV, C, D), jnp.float32),
            pltpu.SemaphoreType.DMA((N_DEV,)),
            pltpu.SemaphoreType.DMA((N_DEV,)),
            pltpu.SemaphoreType.DMA((N_DEV,)),
            pltpu.SemaphoreType.DMA((N_DEV,)),
        ],
        compiler_params=pltpu.CompilerParams(collective_id=0),
    )(x, Win0, Wout0, Win1, Wout1, Win2, Wout2)


# device time: 41099 ns/iter; 4.3290x vs baseline; 1.2684x over previous
import jax
import jax.numpy as jnp
from jax import lax
from jax.experimental import pallas as pl
from jax.experimental.pallas import tpu as pltpu

N_DEV = 16
HALF = N_DEV // 2
FAR = tuple(range(N_DEV - 1, HALF - 1, -1))
NEAR = tuple(range(HALF - 1, 0, -1))


def kernel(x, Win0, Wout0, Win1, Wout1, Win2, Wout2):
    B, D = x.shape
    C = B // N_DEV

    def body(x_ref, win0_ref, wout0_ref, win1_ref, wout1_ref,
             win2_ref, wout2_ref, out_ref,
             xbuf0, xg, pbuf, pbf, crecv16, crecv32,
             send_a, recv_a, send_b, recv_b):
        i = lax.axis_index("i")
        f32, bf16 = jnp.float32, jnp.bfloat16

        barrier_sem = pltpu.get_barrier_semaphore()
        for o in range(1, N_DEV):
            pl.semaphore_signal(barrier_sem, inc=1,
                                device_id=((i + o) % N_DEV,),
                                device_id_type=pl.DeviceIdType.MESH)
        pl.semaphore_wait(barrier_sem, N_DEV - 1)

        for q in range(N_DEV):
            xbuf0[pl.ds(q * C, C), :] = x_ref[
                pl.ds(((i - q) % N_DEV) * C, C), :]

        def compute_block(l, win_ref, wout_ref, q0, q1):
            rows = pl.ds(q0 * C, (q1 - q0) * C)
            xr = xbuf0[rows, :] if l == 0 else xg[rows, :].astype(f32)
            h = jnp.maximum(
                jnp.dot(xr, win_ref[...], preferred_element_type=f32), 0.0)
            p = jnp.dot(h, wout_ref[...], preferred_element_type=f32)
            pbuf[rows, :] = p
            if l < 2:
                pbf[rows, :] = p.astype(bf16)

        def rs_send(l, p):
            src, dst = (pbf, crecv16) if l < 2 else (pbuf, crecv32)
            d = pltpu.make_async_remote_copy(
                src_ref=src.at[pl.ds(p * C, C), :],
                dst_ref=dst.at[p],
                send_sem=send_a.at[p],
                recv_sem=recv_a.at[p],
                device_id=((i - p) % N_DEV,),
                device_id_type=pl.DeviceIdType.MESH,
            )
            d.start()
            return d

        def ag_send(o):
            d = pltpu.make_async_remote_copy(
                src_ref=xg.at[pl.ds(0, C), :],
                dst_ref=xg.at[pl.ds(o * C, C), :],
                send_sem=send_b.at[o],
                recv_sem=recv_b.at[o],
                device_id=((i + o) % N_DEV,),
                device_id_type=pl.DeviceIdType.MESH,
            )
            d.start()
            return d

        def ag_wait_recv(o):
            w = pltpu.make_async_remote_copy(
                src_ref=xg.at[pl.ds(o * C, C), :],
                dst_ref=xg.at[pl.ds(o * C, C), :],
                send_sem=send_b.at[o],
                recv_sem=recv_b.at[o],
                device_id=(i,),
                device_id_type=pl.DeviceIdType.MESH,
            )
            w.wait_recv()

        wins = (win0_ref, win1_ref, win2_ref)
        wouts = (wout0_ref, wout1_ref, wout2_ref)

        ag_descs = []
        for l in range(3):
            if l > 0:
                for o in FAR:
                    ag_wait_recv(o)
            compute_block(l, wins[l], wouts[l], HALF, N_DEV)
            rs_descs = {p: rs_send(l, p) for p in FAR}
            if l > 0:
                for o in NEAR:
                    ag_wait_recv(o)
            compute_block(l, wins[l], wouts[l], 0, HALF)
            rs_descs.update({p: rs_send(l, p) for p in NEAR})
            for d in ag_descs:
                d.wait_send()

            crecv = crecv16 if l < 2 else crecv32
            acc = pbuf[pl.ds(0, C), :]
            for p in FAR + NEAR:
                rs_descs[p].wait_recv()
                acc = acc + crecv[p].astype(f32)
            for d in rs_descs.values():
                d.wait_send()

            if l < 2:
                xg[pl.ds(0, C), :] = acc.astype(bf16)
                ag_descs = [ag_send(o) for o in FAR + NEAR]
            else:
                out_ref[...] = acc

    return pl.pallas_call(
        body,
        out_shape=jax.ShapeDtypeStruct((C, D), jnp.float32),
        in_specs=[pl.BlockSpec(memory_space=pltpu.VMEM)] * 7,
        out_specs=pl.BlockSpec(memory_space=pltpu.VMEM),
        scratch_shapes=[
            pltpu.VMEM((B, D), jnp.float32),
            pltpu.VMEM((B, D), jnp.bfloat16),
            pltpu.VMEM((B, D), jnp.float32),
            pltpu.VMEM((B, D), jnp.bfloat16),
            pltpu.VMEM((N_DEV, C, D), jnp.bfloat16),
            pltpu.VMEM((N_DEV, C, D), jnp.float32),
            pltpu.SemaphoreType.DMA((N_DEV,)),
            pltpu.SemaphoreType.DMA((N_DEV,)),
            pltpu.SemaphoreType.DMA((N_DEV,)),
            pltpu.SemaphoreType.DMA((N_DEV,)),
        ],
        compiler_params=pltpu.CompilerParams(collective_id=0),
    )(x, Win0, Wout0, Win1, Wout1, Win2, Wout2)


# device time: 39750 ns/iter; 4.4759x vs baseline; 1.0339x over previous
import jax
import jax.numpy as jnp
from jax import lax
from jax.experimental import pallas as pl
from jax.experimental.pallas import tpu as pltpu

N_DEV = 16
HALF = N_DEV // 2
FAR = tuple(range(N_DEV - 1, HALF - 1, -1))
NEAR = tuple(range(HALF - 1, 0, -1))


def kernel(x, Win0, Wout0, Win1, Wout1, Win2, Wout2):
    B, D = x.shape
    C = B // N_DEV

    def body(x_ref, win0_ref, wout0_ref, win1_ref, wout1_ref,
             win2_ref, wout2_ref, out_ref,
             xbuf0, xg, pbuf, pbf, crecv16,
             send_a, recv_a, send_b, recv_b):
        i = lax.axis_index("i")
        f32, bf16 = jnp.float32, jnp.bfloat16

        barrier_sem = pltpu.get_barrier_semaphore()
        for o in range(1, N_DEV):
            pl.semaphore_signal(barrier_sem, inc=1,
                                device_id=((i + o) % N_DEV,),
                                device_id_type=pl.DeviceIdType.MESH)
        pl.semaphore_wait(barrier_sem, N_DEV - 1)

        for q in range(N_DEV):
            xbuf0[pl.ds(q * C, C), :] = x_ref[
                pl.ds(((i - q) % N_DEV) * C, C), :]

        def compute_block(l, win_ref, wout_ref, q0, q1):
            rows = pl.ds(q0 * C, (q1 - q0) * C)
            xr = xbuf0[rows, :] if l == 0 else xg[rows, :].astype(f32)
            h = jnp.maximum(
                jnp.dot(xr, win_ref[...], preferred_element_type=f32), 0.0)
            p = jnp.dot(h, wout_ref[...], preferred_element_type=f32)
            pbuf[rows, :] = p
            pbf[rows, :] = p.astype(bf16)

        def rs_send(l, p):
            d = pltpu.make_async_remote_copy(
                src_ref=pbf.at[pl.ds(p * C, C), :],
                dst_ref=crecv16.at[p],
                send_sem=send_a.at[p],
                recv_sem=recv_a.at[p],
                device_id=((i - p) % N_DEV,),
                device_id_type=pl.DeviceIdType.MESH,
            )
            d.start()
            return d

        def ag_send(o):
            d = pltpu.make_async_remote_copy(
                src_ref=xg.at[pl.ds(0, C), :],
                dst_ref=xg.at[pl.ds(o * C, C), :],
                send_sem=send_b.at[o],
                recv_sem=recv_b.at[o],
                device_id=((i + o) % N_DEV,),
                device_id_type=pl.DeviceIdType.MESH,
            )
            d.start()
            return d

        def ag_wait_recv(o):
            w = pltpu.make_async_remote_copy(
                src_ref=xg.at[pl.ds(o * C, C), :],
                dst_ref=xg.at[pl.ds(o * C, C), :],
                send_sem=send_b.at[o],
                recv_sem=recv_b.at[o],
                device_id=(i,),
                device_id_type=pl.DeviceIdType.MESH,
            )
            w.wait_recv()

        wins = (win0_ref, win1_ref, win2_ref)
        wouts = (wout0_ref, wout1_ref, wout2_ref)

        ag_descs = []
        for l in range(3):
            if l > 0:
                for o in FAR:
                    ag_wait_recv(o)
            compute_block(l, wins[l], wouts[l], HALF, N_DEV)
            rs_descs = {p: rs_send(l, p) for p in FAR}
            if l > 0:
                for o in NEAR:
                    ag_wait_recv(o)
            compute_block(l, wins[l], wouts[l], 0, HALF)
            rs_descs.update({p: rs_send(l, p) for p in NEAR})
            for d in ag_descs:
                d.wait_send()

            acc = pbuf[pl.ds(0, C), :]
            for p in FAR + NEAR:
                rs_descs[p].wait_recv()
                acc = acc + crecv16[p].astype(f32)
            for d in rs_descs.values():
                d.wait_send()

            if l < 2:
                xg[pl.ds(0, C), :] = acc.astype(bf16)
                ag_descs = [ag_send(o) for o in FAR + NEAR]
            else:
                out_ref[...] = acc

    return pl.pallas_call(
        body,
        out_shape=jax.ShapeDtypeStruct((C, D), jnp.float32),
        in_specs=[pl.BlockSpec(memory_space=pltpu.VMEM)] * 7,
        out_specs=pl.BlockSpec(memory_space=pltpu.VMEM),
        scratch_shapes=[
            pltpu.VMEM((B, D), jnp.float32),
            pltpu.VMEM((B, D), jnp.bfloat16),
            pltpu.VMEM((B, D), jnp.float32),
            pltpu.VMEM((B, D), jnp.bfloat16),
            pltpu.VMEM((N_DEV, C, D), jnp.bfloat16),
            pltpu.SemaphoreType.DMA((N_DEV,)),
            pltpu.SemaphoreType.DMA((N_DEV,)),
            pltpu.SemaphoreType.DMA((N_DEV,)),
            pltpu.SemaphoreType.DMA((N_DEV,)),
        ],
        compiler_params=pltpu.CompilerParams(collective_id=0),
    )(x, Win0, Wout0, Win1, Wout1, Win2, Wout2)
